# Initial kernel scaffold; baseline (speedup 1.0000x reference)
import jax
import jax.numpy as jnp
from jax.experimental import pallas as pl


def kernel(x, Wq, Wk, Wv, Wo):
    raise NotImplementedError("write your pallas kernel here")


# speedup vs baseline: 1.1286x; 1.1286x over previous
import functools
import math

import jax
import jax.numpy as jnp
from jax.experimental import pallas as pl

_TILE = 16
_MASK_VAL = -10000000000.0


def _kv_kernel(xs_ref, wk_ref, wv_ref, ks_ref, vs_ref):
    xs = xs_ref[0]
    ks_ref[0, 0] = jnp.dot(xs, wk_ref[0], preferred_element_type=jnp.float32)
    vs_ref[0, 0] = jnp.dot(xs, wv_ref[0], preferred_element_type=jnp.float32)


def _attn_kernel(x_ref, wq_ref, ks_ref, vs_ref, wo_ref, out_ref,
                 *, tile_q, seq_len, heads, head_dim, n_keys):
    t = pl.program_id(1)
    x_tile = x_ref[0]
    q = jnp.dot(x_tile, wq_ref[...], preferred_element_type=jnp.float32)
    scale = 1.0 / math.sqrt(head_dim)

    q_idx = t * tile_q + jax.lax.broadcasted_iota(jnp.int32, (tile_q, n_keys), 0)
    k_io = jax.lax.broadcasted_iota(jnp.int32, (tile_q, n_keys), 1)
    last_tile_start = ((seq_len - 1) // _TILE) * _TILE
    k_tok = jnp.where(k_io < _TILE, k_io, k_io - _TILE + last_tile_start)
    future = k_tok > q_idx

    outs = []
    for h in range(heads):
        qh = q[:, h * head_dim:(h + 1) * head_dim]
        kh = ks_ref[0, h]
        vh = vs_ref[0, h]
        logits = jax.lax.dot_general(
            qh, kh, (((1,), (1,)), ((), ())),
            preferred_element_type=jnp.float32) * scale
        logits = jnp.where(future, _MASK_VAL, logits)
        m = jnp.max(logits, axis=-1, keepdims=True)
        e = jnp.exp(logits - m)
        w = e / jnp.sum(e, axis=-1, keepdims=True)
        outs.append(jnp.dot(w, vh, preferred_element_type=jnp.float32))
    attn = jnp.concatenate(outs, axis=-1)
    out_ref[0] = jnp.dot(attn, wo_ref[...], preferred_element_type=jnp.float32)


@jax.jit
def kernel(x, Wq, Wk, Wv, Wo):
    batch, seq_len, d_model = x.shape
    heads = Wq.shape[1] // 64
    head_dim = Wq.shape[1] // heads
    tile_q = 512
    last_tile_start = ((seq_len - 1) // _TILE) * _TILE
    n_keys = 2 * _TILE

    xs = jnp.concatenate(
        [x[:, :_TILE, :], x[:, last_tile_start:last_tile_start + _TILE, :]],
        axis=1)

    wk_h = Wk.reshape(d_model, heads, head_dim).transpose(1, 0, 2)
    wv_h = Wv.reshape(d_model, heads, head_dim).transpose(1, 0, 2)

    ks, vs = pl.pallas_call(
        _kv_kernel,
        grid=(batch, heads),
        in_specs=[
            pl.BlockSpec((1, n_keys, d_model), lambda b, h: (b, 0, 0)),
            pl.BlockSpec((1, d_model, head_dim), lambda b, h: (h, 0, 0)),
            pl.BlockSpec((1, d_model, head_dim), lambda b, h: (h, 0, 0)),
        ],
        out_specs=[
            pl.BlockSpec((1, 1, n_keys, head_dim), lambda b, h: (b, h, 0, 0)),
            pl.BlockSpec((1, 1, n_keys, head_dim), lambda b, h: (b, h, 0, 0)),
        ],
        out_shape=[
            jax.ShapeDtypeStruct((batch, heads, n_keys, head_dim), jnp.float32),
            jax.ShapeDtypeStruct((batch, heads, n_keys, head_dim), jnp.float32),
        ],
    )(xs, wk_h, wv_h)

    out = pl.pallas_call(
        functools.partial(
            _attn_kernel, tile_q=tile_q, seq_len=seq_len, heads=heads,
            head_dim=head_dim, n_keys=n_keys),
        grid=(batch, seq_len // tile_q),
        in_specs=[
            pl.BlockSpec((1, tile_q, d_model), lambda b, t: (b, t, 0)),
            pl.BlockSpec((d_model, heads * head_dim), lambda b, t: (0, 0)),
            pl.BlockSpec((1, heads, n_keys, head_dim), lambda b, t: (b, 0, 0, 0)),
            pl.BlockSpec((1, heads, n_keys, head_dim), lambda b, t: (b, 0, 0, 0)),
            pl.BlockSpec((heads * head_dim, d_model), lambda b, t: (0, 0)),
        ],
        out_specs=pl.BlockSpec((1, tile_q, d_model), lambda b, t: (b, t, 0)),
        out_shape=jax.ShapeDtypeStruct((batch, seq_len, d_model), jnp.float32),
    )(x, Wq, ks, vs, Wo)
    return out


# speedup vs baseline: 3.0005x; 2.6587x over previous
import functools
import math

import jax
import jax.numpy as jnp
from jax.experimental import pallas as pl

_TILE = 16
_MASK_VAL = -10000000000.0


def _prep_kernel(xs_ref, xst_ref, wq_ref, wkt_ref, wv_ref, wo_ref,
                 qk_ref, vo_ref, *, heads, head_dim, n_keys):
    d_model = wq_ref.shape[0]
    kst = jnp.dot(wkt_ref[...], xst_ref[0], preferred_element_type=jnp.float32)
    kbd = jnp.concatenate([kst] * heads, axis=1)
    row_h = jax.lax.broadcasted_iota(jnp.int32, (d_model, heads * n_keys), 0) // head_dim
    col_h = jax.lax.broadcasted_iota(jnp.int32, (d_model, heads * n_keys), 1) // n_keys
    kbd = jnp.where(row_h == col_h, kbd, 0.0)
    scale = 1.0 / math.sqrt(head_dim)
    qk_ref[0] = jnp.dot(wq_ref[...], kbd,
                        preferred_element_type=jnp.float32) * scale

    vsf = jnp.dot(xs_ref[0], wv_ref[...], preferred_element_type=jnp.float32)
    vbd = jnp.concatenate([vsf] * heads, axis=0)
    row_v = jax.lax.broadcasted_iota(jnp.int32, (heads * n_keys, d_model), 0) // n_keys
    col_v = jax.lax.broadcasted_iota(jnp.int32, (heads * n_keys, d_model), 1) // head_dim
    vbd = jnp.where(row_v == col_v, vbd, 0.0)
    vo_ref[0] = jnp.dot(vbd, wo_ref[...], preferred_element_type=jnp.float32)


def _attn_kernel(x_ref, qk_ref, vo_ref, gg_ref, out_ref,
                 *, tile_q, seq_len, n_keys, heads):
    t = pl.program_id(1)
    width = heads * n_keys
    l = jnp.dot(x_ref[0], qk_ref[0], preferred_element_type=jnp.float32)
    q_idx = t * tile_q + jax.lax.broadcasted_iota(jnp.int32, (tile_q, width), 0)
    k_io = jax.lax.broadcasted_iota(jnp.int32, (tile_q, width), 1) % n_keys
    last_tile_start = ((seq_len - 1) // _TILE) * _TILE
    k_tok = jnp.where(k_io < _TILE, k_io, k_io - _TILE + last_tile_start)
    l = jnp.where(k_tok > q_idx, _MASK_VAL, l)
    m = jnp.max(l, axis=-1, keepdims=True)
    e = jnp.exp(l - m)
    d = jnp.dot(e, gg_ref[...], preferred_element_type=jnp.float32)
    out_ref[0] = jnp.dot(e / d, vo_ref[0], preferred_element_type=jnp.float32)


@jax.jit
def kernel(x, Wq, Wk, Wv, Wo):
    batch, seq_len, d_model = x.shape
    head_dim = 64
    heads = Wq.shape[1] // head_dim
    tile_q = 512
    last_tile_start = ((seq_len - 1) // _TILE) * _TILE
    n_keys = 2 * _TILE
    width = heads * n_keys

    xs = jnp.concatenate(
        [x[:, :_TILE, :], x[:, last_tile_start:last_tile_start + _TILE, :]],
        axis=1)
    xst = jnp.transpose(xs, (0, 2, 1))

    qk, vo = pl.pallas_call(
        functools.partial(_prep_kernel, heads=heads, head_dim=head_dim,
                          n_keys=n_keys),
        grid=(batch,),
        in_specs=[
            pl.BlockSpec((1, n_keys, d_model), lambda b: (b, 0, 0)),
            pl.BlockSpec((1, d_model, n_keys), lambda b: (b, 0, 0)),
            pl.BlockSpec((d_model, heads * head_dim), lambda b: (0, 0)),
            pl.BlockSpec((d_model, heads * head_dim), lambda b: (0, 0)),
            pl.BlockSpec((d_model, heads * head_dim), lambda b: (0, 0)),
            pl.BlockSpec((heads * head_dim, d_model), lambda b: (0, 0)),
        ],
        out_specs=[
            pl.BlockSpec((1, d_model, width), lambda b: (b, 0, 0)),
            pl.BlockSpec((1, width, d_model), lambda b: (b, 0, 0)),
        ],
        out_shape=[
            jax.ShapeDtypeStruct((batch, d_model, width), jnp.float32),
            jax.ShapeDtypeStruct((batch, width, d_model), jnp.float32),
        ],
    )(xs, xst, Wq, Wk.T, Wv, Wo)

    gg = (jnp.arange(width)[:, None] // n_keys ==
          jnp.arange(width)[None, :] // n_keys).astype(jnp.float32)

    out = pl.pallas_call(
        functools.partial(_attn_kernel, tile_q=tile_q, seq_len=seq_len,
                          n_keys=n_keys, heads=heads),
        grid=(batch, seq_len // tile_q),
        in_specs=[
            pl.BlockSpec((1, tile_q, d_model), lambda b, t: (b, t, 0)),
            pl.BlockSpec((1, d_model, width), lambda b, t: (b, 0, 0)),
            pl.BlockSpec((1, width, d_model), lambda b, t: (b, 0, 0)),
            pl.BlockSpec((width, width), lambda b, t: (0, 0)),
        ],
        out_specs=pl.BlockSpec((1, tile_q, d_model), lambda b, t: (b, t, 0)),
        out_shape=jax.ShapeDtypeStruct((batch, seq_len, d_model), jnp.float32),
    )(x, qk, vo, gg)
    return out
